# baseline (speedup 1.0000x reference)
import jax
import jax.numpy as jnp
from jax.experimental import pallas as pl
from jax.experimental.pallas import tpu as pltpu


def _body(pt_ref, tt_ref, out_ref):
    out_ref[...] = (pt_ref[0:1, 0:1] + tt_ref[0:1, 0:1])


@jax.jit
def run(pred, target):
    pt = pred
    tt = target
    out = pl.pallas_call(
        _body,
        out_shape=jax.ShapeDtypeStruct((1, 1), jnp.float32),
        in_specs=[pl.BlockSpec(memory_space=pltpu.VMEM),
                  pl.BlockSpec(memory_space=pltpu.VMEM)],
        out_specs=pl.BlockSpec(memory_space=pltpu.VMEM),
    )(pt, tt)
    return out[0, 0]


def kernel(pred, target, cls):
    return run(pred, target)


# speedup vs baseline: 4.0313x; 4.0313x over previous
import jax
import jax.numpy as jnp
from jax.experimental import pallas as pl
from jax.experimental.pallas import tpu as pltpu

_N = 20000


def _tc_body(st_ref, out_ref):
    p0 = st_ref[0:1, :]
    p1 = st_ref[1:2, :]
    p2 = st_ref[2:3, :]
    p3 = st_ref[3:4, :]
    p4 = st_ref[4:5, :]
    t0 = st_ref[5:6, :]
    t1 = st_ref[6:7, :]
    t2 = st_ref[7:8, :]
    t3 = st_ref[8:9, :]
    t4 = st_ref[9:10, :]

    ad01 = jnp.abs(p0 - t0) + jnp.abs(p1 - t1)
    ad2 = jnp.abs(p2 - t2)
    ad3 = jnp.abs(p3 - t3)
    ad4 = jnp.abs(p4 - t4)

    e = jnp.abs(p2 - p3) > 0.5
    ew = jnp.where(e, 1.0, 0.0)
    cw = 1.0 - ew

    e_sum = jnp.sum(ew * (ad01 + ad2 + ad3 + ad4), keepdims=True)
    c_sum = jnp.sum(cw * (ad01 + jnp.abs(p2 + p3 - 2.0 * t2) + jnp.abs(t4)),
                    keepdims=True)
    ne = jnp.sum(ew, keepdims=True)
    nc = jnp.float32(_N) - ne

    out_ref[...] = (e_sum / jnp.maximum(ne, 1.0)
                    + c_sum / jnp.maximum(nc, 1.0))


@jax.jit
def tc_loss(pred, target):
    st = jnp.concatenate([pred, target], axis=1).T
    out = pl.pallas_call(
        _tc_body,
        out_shape=jax.ShapeDtypeStruct((1, 1), jnp.float32),
        in_specs=[pl.BlockSpec(memory_space=pltpu.VMEM)],
        out_specs=pl.BlockSpec(memory_space=pltpu.VMEM),
    )(st)
    return out[0, 0]


def kernel(pred, target, cls):
    return tc_loss(pred, target)


# speedup vs baseline: 5.7300x; 1.4214x over previous
import jax
import jax.numpy as jnp
from jax.experimental import pallas as pl
from jax.experimental.pallas import tpu as pltpu

_N = 20000
_STEPS = 4
_CHUNK = 5120


def _tc_body(pt_ref, tt_ref, out_ref, acc_ref):
    i = pl.program_id(0)

    @pl.when(i == 0)
    def _():
        acc_ref[0] = 0.0
        acc_ref[1] = 0.0
        acc_ref[2] = 0.0

    p0 = pt_ref[0:1, :]
    p1 = pt_ref[1:2, :]
    p2 = pt_ref[2:3, :]
    p3 = pt_ref[3:4, :]
    p4 = pt_ref[4:5, :]
    t0 = tt_ref[0:1, :]
    t1 = tt_ref[1:2, :]
    t2 = tt_ref[2:3, :]
    t3 = tt_ref[3:4, :]
    t4 = tt_ref[4:5, :]

    ad01 = jnp.abs(p0 - t0) + jnp.abs(p1 - t1)
    ad2 = jnp.abs(p2 - t2)
    ad3 = jnp.abs(p3 - t3)
    ad4 = jnp.abs(p4 - t4)

    lane = jax.lax.broadcasted_iota(jnp.int32, (1, _CHUNK), 1)
    valid = (i * _CHUNK + lane) < _N
    e = (jnp.abs(p2 - p3) > 0.5) & valid
    ew = jnp.where(e, 1.0, 0.0)
    cw = jnp.where(valid, 1.0, 0.0) - ew

    e_sum = jnp.sum(ew * (ad01 + ad2 + ad3 + ad4))
    c_sum = jnp.sum(cw * (ad01 + jnp.abs(p2 + p3 - 2.0 * t2) + jnp.abs(t4)))
    ne = jnp.sum(ew)

    acc_ref[0] = acc_ref[0] + e_sum
    acc_ref[1] = acc_ref[1] + c_sum
    acc_ref[2] = acc_ref[2] + ne

    @pl.when(i == _STEPS - 1)
    def _():
        ne_t = acc_ref[2]
        nc_t = jnp.float32(_N) - ne_t
        res = (acc_ref[0] / jnp.maximum(ne_t, 1.0)
               + acc_ref[1] / jnp.maximum(nc_t, 1.0))
        out_ref[...] = jnp.full((1, 1), res, jnp.float32)


@jax.jit
def tc_loss(pred, target):
    out = pl.pallas_call(
        _tc_body,
        grid=(_STEPS,),
        out_shape=jax.ShapeDtypeStruct((1, 1), jnp.float32),
        in_specs=[pl.BlockSpec((5, _CHUNK), lambda i: (0, i)),
                  pl.BlockSpec((5, _CHUNK), lambda i: (0, i))],
        out_specs=pl.BlockSpec((1, 1), lambda i: (0, 0)),
        scratch_shapes=[pltpu.SMEM((3,), jnp.float32)],
    )(pred.T, target.T)
    return out[0, 0]


def kernel(pred, target, cls):
    return tc_loss(pred, target)


# speedup vs baseline: 8.2008x; 1.4312x over previous
import jax
import jax.numpy as jnp
from jax.experimental import pallas as pl
from jax.experimental.pallas import tpu as pltpu

_N = 20000


def _tc_body(pt_ref, tt_ref, out_ref):
    p0 = pt_ref[0:1, :]
    p1 = pt_ref[1:2, :]
    p2 = pt_ref[2:3, :]
    p3 = pt_ref[3:4, :]
    p4 = pt_ref[4:5, :]
    t0 = tt_ref[0:1, :]
    t1 = tt_ref[1:2, :]
    t2 = tt_ref[2:3, :]
    t3 = tt_ref[3:4, :]
    t4 = tt_ref[4:5, :]

    ad01 = jnp.abs(p0 - t0) + jnp.abs(p1 - t1)
    ad2 = jnp.abs(p2 - t2)
    ad3 = jnp.abs(p3 - t3)
    ad4 = jnp.abs(p4 - t4)

    e = jnp.abs(p2 - p3) > 0.5
    ew = jnp.where(e, 1.0, 0.0)
    cw = 1.0 - ew

    e_sum = jnp.sum(ew * (ad01 + ad2 + ad3 + ad4), keepdims=True)
    c_sum = jnp.sum(cw * (ad01 + jnp.abs(p2 + p3 - 2.0 * t2) + jnp.abs(t4)),
                    keepdims=True)
    ne = jnp.sum(ew, keepdims=True)
    nc = jnp.float32(_N) - ne

    out_ref[...] = (e_sum / jnp.maximum(ne, 1.0)
                    + c_sum / jnp.maximum(nc, 1.0))


@jax.jit
def tc_loss(pred, target):
    pt = pred.T
    tt = target.T
    out = pl.pallas_call(
        _tc_body,
        out_shape=jax.ShapeDtypeStruct((1, 1), jnp.float32),
        in_specs=[pl.BlockSpec(memory_space=pltpu.VMEM),
                  pl.BlockSpec(memory_space=pltpu.VMEM)],
        out_specs=pl.BlockSpec(memory_space=pltpu.VMEM),
    )(pt, tt)
    return out[0, 0]


def kernel(pred, target, cls):
    return tc_loss(pred, target)
